# Initial kernel scaffold; baseline (speedup 1.0000x reference)
import jax
import jax.numpy as jnp
from jax.experimental import pallas as pl


def kernel(hidden_states, router_weight, e_score_correction_bias, w13_weight, w2_weight):
    raise NotImplementedError("write your pallas kernel here")


# speedup vs baseline: 1.3774x; 1.3774x over previous
import jax
import jax.numpy as jnp
from jax import lax
from jax.experimental import pallas as pl
from jax.experimental.pallas import tpu as pltpu

T = 64
H = 1024
F = 512
E = 64
ZE = 16
K = 2
SCALE = 2.5


def _moe_kernel(x_ref, wr_ref, bias_ref, w13_ref, w2_ref, out_ref, ids_s, wts_s):
    e = pl.program_id(0)

    @pl.when(e == 0)
    def _router():
        x = x_ref[...]
        logits = lax.dot_general(x, wr_ref[...], (((1,), (1,)), ((), ())),
                                 preferred_element_type=jnp.float32)
        s = jax.nn.softmax(logits, axis=-1)
        sc = s + bias_ref[...]
        iota = lax.broadcasted_iota(jnp.int32, (T, E + ZE), 1)
        m1 = jnp.max(sc, axis=1, keepdims=True)
        i1 = jnp.min(jnp.where(sc == m1, iota, E + ZE), axis=1, keepdims=True)
        w1 = jnp.sum(jnp.where(iota == i1, s, 0.0), axis=1, keepdims=True) * SCALE
        sc2 = jnp.where(iota == i1, -jnp.inf, sc)
        m2 = jnp.max(sc2, axis=1, keepdims=True)
        i2 = jnp.min(jnp.where(sc2 == m2, iota, E + ZE), axis=1, keepdims=True)
        w2 = jnp.sum(jnp.where(iota == i2, s, 0.0), axis=1, keepdims=True) * SCALE
        zsc = (jnp.where(i1 >= E, w1, 0.0) + jnp.where(i2 >= E, w2, 0.0))
        out_ref[...] = x * zsc
        ids_s[:, 0:1] = i1
        ids_s[:, 1:2] = i2
        wts_s[:, 0:1] = w1
        wts_s[:, 1:2] = w2

    x = x_ref[...]
    h13 = lax.dot_general(x, w13_ref[0], (((1,), (1,)), ((), ())),
                          preferred_element_type=jnp.float32)
    gate = h13[:, :F]
    up = h13[:, F:]
    act = gate * jax.nn.sigmoid(gate) * up
    oute = lax.dot_general(act, w2_ref[0], (((1,), (1,)), ((), ())),
                           preferred_element_type=jnp.float32)
    wcol = (jnp.where(ids_s[:, 0:1] == e, wts_s[:, 0:1], 0.0)
            + jnp.where(ids_s[:, 1:2] == e, wts_s[:, 1:2], 0.0))
    out_ref[...] += wcol * oute


def kernel(hidden_states, router_weight, e_score_correction_bias, w13_weight, w2_weight):
    bias2d = e_score_correction_bias.reshape(1, E + ZE)
    return pl.pallas_call(
        _moe_kernel,
        grid=(E,),
        in_specs=[
            pl.BlockSpec((T, H), lambda e: (0, 0)),
            pl.BlockSpec((E + ZE, H), lambda e: (0, 0)),
            pl.BlockSpec((1, E + ZE), lambda e: (0, 0)),
            pl.BlockSpec((1, 2 * F, H), lambda e: (e, 0, 0)),
            pl.BlockSpec((1, H, F), lambda e: (e, 0, 0)),
        ],
        out_specs=pl.BlockSpec((T, H), lambda e: (0, 0)),
        out_shape=jax.ShapeDtypeStruct((T, H), jnp.float32),
        scratch_shapes=[
            pltpu.VMEM((T, 128), jnp.int32),
            pltpu.VMEM((T, 128), jnp.float32),
        ],
    )(hidden_states, router_weight, bias2d, w13_weight, w2_weight)


# speedup vs baseline: 1.6533x; 1.2003x over previous
import jax
import jax.numpy as jnp
from jax import lax
from jax.experimental import pallas as pl
from jax.experimental.pallas import tpu as pltpu

T = 64
H = 1024
F = 512
E = 64
ZE = 16
K = 2
SCALE = 2.5
NC = E + ZE


def _router_kernel(x_ref, wr_ref, bias_ref, ids_ref, wts_ref, sched_ref):
    x = x_ref[...]
    logits = lax.dot_general(x, wr_ref[...], (((1,), (1,)), ((), ())),
                             preferred_element_type=jnp.float32)
    s = jax.nn.softmax(logits, axis=-1)
    sc = s + bias_ref[...]
    iota = lax.broadcasted_iota(jnp.int32, (T, NC), 1)
    m1 = jnp.max(sc, axis=1, keepdims=True)
    i1 = jnp.min(jnp.where(sc == m1, iota, NC), axis=1, keepdims=True)
    w1 = jnp.sum(jnp.where(iota == i1, s, 0.0), axis=1, keepdims=True) * SCALE
    sc2 = jnp.where(iota == i1, -jnp.inf, sc)
    m2 = jnp.max(sc2, axis=1, keepdims=True)
    i2 = jnp.min(jnp.where(sc2 == m2, iota, NC), axis=1, keepdims=True)
    w2 = jnp.sum(jnp.where(iota == i2, s, 0.0), axis=1, keepdims=True) * SCALE
    zsc = jnp.where(i1 >= E, w1, 0.0) + jnp.where(i2 >= E, w2, 0.0)

    ids_ref[...] = jnp.zeros((T, 128), jnp.int32)
    ids_ref[:, 0:1] = i1
    ids_ref[:, 1:2] = i2
    wts_ref[...] = jnp.zeros((T, 128), jnp.float32)
    wts_ref[:, 0:1] = w1
    wts_ref[:, 1:2] = w2
    wts_ref[:, 2:3] = zsc

    eio = lax.broadcasted_iota(jnp.int32, (T, E), 1)
    hitf = ((i1 == eio) | (i2 == eio)).astype(jnp.float32)
    ones_col = jnp.ones((T, 1), jnp.float32)
    nhit = lax.dot_general(hitf, ones_col, (((0,), (0,)), ((), ())),
                           preferred_element_type=jnp.float32)
    activef = (nhit > 0.0).astype(jnp.float32)
    io0 = lax.broadcasted_iota(jnp.int32, (E, E), 0)
    io1 = lax.broadcasted_iota(jnp.int32, (E, E), 1)
    ltri = (io0 >= io1).astype(jnp.float32)
    rank = lax.dot_general(ltri, activef, (((1,), (0,)), ((), ())),
                           preferred_element_type=jnp.float32)
    jrow = lax.broadcasted_iota(jnp.int32, (1, E), 1)
    onehot = ((rank.astype(jnp.int32) == jrow + 1) &
              (activef > 0.0)).astype(jnp.float32)
    erow = lax.broadcasted_iota(jnp.int32, (1, E), 1).astype(jnp.float32)
    sched_row = lax.dot_general(erow, onehot, (((1,), (0,)), ((), ())),
                                preferred_element_type=jnp.float32)
    count = jnp.sum(activef, axis=0, keepdims=True).astype(jnp.int32)
    ecol = lax.broadcasted_iota(jnp.int32, (E, 1), 0)
    last = jnp.max(jnp.where(activef > 0.0, ecol, -1), axis=0, keepdims=True)
    sched = jnp.where(jrow < count, sched_row.astype(jnp.int32),
                      jnp.maximum(last, 0))
    sched_ref[...] = jnp.zeros((8, 128), jnp.int32)
    sched_ref[0:1, 0:E] = sched
    sched_ref[0:1, E:E + 1] = count


def _expert_kernel(sched_ref, count_ref, x_ref, ids_ref, wts_ref,
                   w13_ref, w2_ref, out_ref):
    i = pl.program_id(0)
    e = sched_ref[i]

    @pl.when(i == 0)
    def _init():
        out_ref[...] = x_ref[...] * wts_ref[:, 2:3]

    @pl.when(i < count_ref[0])
    def _expert():
        x = x_ref[...]
        h13 = lax.dot_general(x, w13_ref[0], (((1,), (1,)), ((), ())),
                              preferred_element_type=jnp.float32)
        gate = h13[:, :F]
        up = h13[:, F:]
        act = gate * jax.nn.sigmoid(gate) * up
        oute = lax.dot_general(act, w2_ref[0], (((1,), (1,)), ((), ())),
                               preferred_element_type=jnp.float32)
        wcol = (jnp.where(ids_ref[:, 0:1] == e, wts_ref[:, 0:1], 0.0)
                + jnp.where(ids_ref[:, 1:2] == e, wts_ref[:, 1:2], 0.0))
        out_ref[...] += wcol * oute


def kernel(hidden_states, router_weight, e_score_correction_bias, w13_weight, w2_weight):
    bias2d = e_score_correction_bias.reshape(1, NC)
    ids, wts, sched2d = pl.pallas_call(
        _router_kernel,
        in_specs=[
            pl.BlockSpec((T, H), lambda: (0, 0)),
            pl.BlockSpec((NC, H), lambda: (0, 0)),
            pl.BlockSpec((1, NC), lambda: (0, 0)),
        ],
        out_specs=[
            pl.BlockSpec((T, 128), lambda: (0, 0)),
            pl.BlockSpec((T, 128), lambda: (0, 0)),
            pl.BlockSpec((8, 128), lambda: (0, 0)),
        ],
        out_shape=[
            jax.ShapeDtypeStruct((T, 128), jnp.int32),
            jax.ShapeDtypeStruct((T, 128), jnp.float32),
            jax.ShapeDtypeStruct((8, 128), jnp.int32),
        ],
    )(hidden_states, router_weight, bias2d)

    sched = sched2d[0, 0:E]
    count = sched2d[0, E:E + 1]

    grid_spec = pltpu.PrefetchScalarGridSpec(
        num_scalar_prefetch=2,
        grid=(E,),
        in_specs=[
            pl.BlockSpec((T, H), lambda i, s, c: (0, 0)),
            pl.BlockSpec((T, 128), lambda i, s, c: (0, 0)),
            pl.BlockSpec((T, 128), lambda i, s, c: (0, 0)),
            pl.BlockSpec((1, 2 * F, H), lambda i, s, c: (s[i], 0, 0)),
            pl.BlockSpec((1, H, F), lambda i, s, c: (s[i], 0, 0)),
        ],
        out_specs=pl.BlockSpec((T, H), lambda i, s, c: (0, 0)),
    )
    return pl.pallas_call(
        _expert_kernel,
        grid_spec=grid_spec,
        out_shape=jax.ShapeDtypeStruct((T, H), jnp.float32),
    )(sched, count, hidden_states, ids, wts, w13_weight, w2_weight)
